# baseline (speedup 1.0000x reference)
import functools

import jax
import jax.numpy as jnp
from jax.experimental import pallas as pl
from jax.experimental.pallas import tpu as pltpu


def _ffn_block_kernel(e_map_ref, active_ref, xs_ref, w1_ref, w2_ref, out_ref,
                      acc_ref, *, nf):
    f = pl.program_id(1)

    @pl.when(active_ref[pl.program_id(0)] > 0)
    def _():
        x = xs_ref[...]
        w1 = w1_ref[0].astype(jnp.bfloat16)
        h = jax.lax.dot_general(
            x, w1, (((1,), (1,)), ((), ())),
            preferred_element_type=jnp.float32)
        h = jnp.maximum(h, 0.0)
        h = (h * h).astype(jnp.bfloat16)
        w2 = w2_ref[0].astype(jnp.bfloat16)
        acc = jax.lax.dot_general(
            h, w2, (((1,), (1,)), ((), ())),
            preferred_element_type=jnp.float32)

        @pl.when(f == 0)
        def _():
            acc_ref[...] = acc

        @pl.when(f != 0)
        def _():
            acc_ref[...] += acc

        @pl.when(f == nf - 1)
        def _():
            out_ref[...] = acc_ref[...].astype(jnp.bfloat16)


@functools.partial(jax.jit, static_argnames=("bm", "f_blk", "d_blk"))
def _moe(x, Wr, W1, W2, bm, f_blk, d_blk):
    B, T, D = x.shape
    E, F, _ = W1.shape
    K = 2
    N = B * T
    NP = N * K

    x_flat = x.reshape(N, D)

    logits = x_flat @ Wr.T
    eids = jax.lax.broadcasted_iota(jnp.int32, (N, E), 1)
    i1 = jnp.argmax(logits, axis=-1).astype(jnp.int32)
    m1 = jnp.max(logits, axis=-1)
    masked = jnp.where(eids == i1[:, None], -jnp.inf, logits)
    i2 = jnp.argmax(masked, axis=-1).astype(jnp.int32)
    m2 = jnp.max(masked, axis=-1)
    g1 = jax.nn.sigmoid(m1 - m2)

    e_pair = jnp.stack([i1, i2], axis=-1).reshape(-1)
    t_pair = jnp.repeat(jnp.arange(N, dtype=jnp.int32), K)
    oh = (e_pair[:, None] == jnp.arange(E, dtype=jnp.int32)[None, :]
          ).astype(jnp.int32)
    cum = jnp.cumsum(oh, axis=0)
    counts = cum[-1]
    rank = jnp.sum((cum - oh) * oh, axis=1)

    padded_counts = ((counts + bm - 1) // bm) * bm
    padded_cum = jnp.cumsum(padded_counts)
    padded_offsets = padded_cum - padded_counts
    slot = padded_offsets[e_pair].astype(jnp.int32) + rank

    nb = NP // bm + E
    total = nb * bm

    src_tok = jnp.zeros((total,), jnp.int32).at[slot].set(t_pair)

    block_base = jnp.arange(nb, dtype=jnp.int32) * bm
    e_of_b = jnp.searchsorted(padded_cum, block_base, side="right").astype(jnp.int32)
    e_of_b = jnp.minimum(e_of_b, E - 1)
    active = (block_base < padded_cum[-1]).astype(jnp.int32)

    xs = jnp.take(x_flat.astype(jnp.bfloat16), src_tok, axis=0)

    nf = F // f_blk
    del d_blk
    grid_spec = pltpu.PrefetchScalarGridSpec(
        num_scalar_prefetch=2,
        grid=(nb, nf),
        in_specs=[
            pl.BlockSpec((bm, D), lambda b, f, e_map, act: (b, 0)),
            pl.BlockSpec((1, f_blk, D), lambda b, f, e_map, act: (e_map[b], f, 0)),
            pl.BlockSpec((1, D, f_blk), lambda b, f, e_map, act: (e_map[b], 0, f)),
        ],
        out_specs=pl.BlockSpec((bm, D), lambda b, f, e_map, act: (b, 0)),
        scratch_shapes=[pltpu.VMEM((bm, D), jnp.float32)],
    )
    eo = pl.pallas_call(
        functools.partial(_ffn_block_kernel, nf=nf),
        grid_spec=grid_spec,
        out_shape=jax.ShapeDtypeStruct((total, D), jnp.bfloat16),
        compiler_params=pltpu.CompilerParams(
            dimension_semantics=("arbitrary", "arbitrary"),
        ),
    )(e_of_b, active, xs, W1, W2)

    slot_tok = slot.reshape(N, K)
    g1c = g1[:, None]
    out = eo[slot_tok[:, 0]] * g1c + eo[slot_tok[:, 1]] * (1.0 - g1c)
    return out.reshape(B, T, D)


def kernel(x, Wr, W1, W2):
    return _moe(x, Wr, W1, W2, bm=512, f_blk=1024, d_blk=128)


# speedup vs baseline: 1.1176x; 1.1176x over previous
import functools

import jax
import jax.numpy as jnp
from jax.experimental import pallas as pl
from jax.experimental.pallas import tpu as pltpu


def _ffn_block_kernel(e_map_ref, active_ref, xs_ref, w1_ref, w2_ref, out_ref):
    f = pl.program_id(1)

    @pl.when(active_ref[pl.program_id(0)] > 0)
    def _():
        x = xs_ref[...]
        w1 = w1_ref[0].astype(jnp.bfloat16)
        h = jax.lax.dot_general(
            x, w1, (((1,), (1,)), ((), ())),
            preferred_element_type=jnp.float32)
        h = jnp.maximum(h, 0.0)
        h = (h * h).astype(jnp.bfloat16)
        w2 = w2_ref[0].astype(jnp.bfloat16)
        acc = jax.lax.dot_general(
            h, w2, (((1,), (1,)), ((), ())),
            preferred_element_type=jnp.float32)

        @pl.when(f == 0)
        def _():
            out_ref[...] = acc

        @pl.when(f != 0)
        def _():
            out_ref[...] += acc


@functools.partial(jax.jit, static_argnames=("bm", "f_blk", "d_blk"))
def _moe(x, Wr, W1, W2, bm, f_blk, d_blk):
    B, T, D = x.shape
    E, F, _ = W1.shape
    K = 2
    N = B * T
    NP = N * K

    x_flat = x.reshape(N, D)

    logits = x_flat @ Wr.T
    eids = jax.lax.broadcasted_iota(jnp.int32, (N, E), 1)
    i1 = jnp.argmax(logits, axis=-1).astype(jnp.int32)
    m1 = jnp.max(logits, axis=-1)
    masked = jnp.where(eids == i1[:, None], -jnp.inf, logits)
    i2 = jnp.argmax(masked, axis=-1).astype(jnp.int32)
    m2 = jnp.max(masked, axis=-1)
    g1 = jax.nn.sigmoid(m1 - m2)

    e_pair = jnp.stack([i1, i2], axis=-1).reshape(-1)
    t_pair = jnp.repeat(jnp.arange(N, dtype=jnp.int32), K)
    oh = (e_pair[:, None] == jnp.arange(E, dtype=jnp.int32)[None, :]
          ).astype(jnp.int32)
    cum = jnp.cumsum(oh, axis=0)
    counts = cum[-1]
    rank = jnp.sum((cum - oh) * oh, axis=1)

    padded_counts = ((counts + bm - 1) // bm) * bm
    padded_cum = jnp.cumsum(padded_counts)
    padded_offsets = padded_cum - padded_counts
    slot = padded_offsets[e_pair].astype(jnp.int32) + rank

    nb = NP // bm + E
    total = nb * bm

    src_tok = jnp.zeros((total,), jnp.int32).at[slot].set(t_pair)

    block_base = jnp.arange(nb, dtype=jnp.int32) * bm
    e_of_b = jnp.searchsorted(padded_cum, block_base, side="right").astype(jnp.int32)
    e_of_b = jnp.minimum(e_of_b, E - 1)
    active = (block_base < padded_cum[-1]).astype(jnp.int32)

    xs = jnp.take(x_flat.astype(jnp.bfloat16), src_tok, axis=0)

    nf = F // f_blk
    del d_blk
    grid_spec = pltpu.PrefetchScalarGridSpec(
        num_scalar_prefetch=2,
        grid=(nb, nf),
        in_specs=[
            pl.BlockSpec((bm, D), lambda b, f, e_map, act: (b, 0)),
            pl.BlockSpec((1, f_blk, D), lambda b, f, e_map, act: (e_map[b], f, 0)),
            pl.BlockSpec((1, D, f_blk), lambda b, f, e_map, act: (e_map[b], 0, f)),
        ],
        out_specs=pl.BlockSpec((bm, D), lambda b, f, e_map, act: (b, 0)),
    )
    eo = pl.pallas_call(
        _ffn_block_kernel,
        grid_spec=grid_spec,
        out_shape=jax.ShapeDtypeStruct((total, D), jnp.float32),
        compiler_params=pltpu.CompilerParams(
            dimension_semantics=("arbitrary", "arbitrary"),
        ),
    )(e_of_b, active, xs, W1, W2)

    slot_tok = slot.reshape(N, K)
    g1c = g1[:, None]
    out = eo[slot_tok[:, 0]] * g1c + eo[slot_tok[:, 1]] * (1.0 - g1c)
    return out.reshape(B, T, D)


def kernel(x, Wr, W1, W2):
    return _moe(x, Wr, W1, W2, bm=512, f_blk=1024, d_blk=128)


# speedup vs baseline: 1.1460x; 1.0254x over previous
import functools

import jax
import jax.numpy as jnp
from jax.experimental import pallas as pl
from jax.experimental.pallas import tpu as pltpu


def _ffn_block_kernel(e_map_ref, active_ref, xs_ref, w1_ref, w2_ref, out_ref):
    f = pl.program_id(1)

    @pl.when(active_ref[pl.program_id(0)] > 0)
    def _():
        x = xs_ref[...]
        w1 = w1_ref[0].astype(jnp.bfloat16)
        h = jax.lax.dot_general(
            x, w1, (((1,), (1,)), ((), ())),
            preferred_element_type=jnp.float32)
        h = jnp.maximum(h, 0.0)
        h = (h * h).astype(jnp.bfloat16)
        w2 = w2_ref[0].astype(jnp.bfloat16)
        acc = jax.lax.dot_general(
            h, w2, (((1,), (1,)), ((), ())),
            preferred_element_type=jnp.float32)

        @pl.when(f == 0)
        def _():
            out_ref[...] = acc

        @pl.when(f != 0)
        def _():
            out_ref[...] += acc


@functools.partial(jax.jit, static_argnames=("bm", "f_blk", "d_blk"))
def _moe(x, Wr, W1, W2, bm, f_blk, d_blk):
    B, T, D = x.shape
    E, F, _ = W1.shape
    K = 2
    N = B * T
    NP = N * K

    x_flat = x.reshape(N, D)

    logits = x_flat @ Wr.T
    eids = jax.lax.broadcasted_iota(jnp.int32, (N, E), 1)
    i1 = jnp.argmax(logits, axis=-1).astype(jnp.int32)
    m1 = jnp.max(logits, axis=-1)
    masked = jnp.where(eids == i1[:, None], -jnp.inf, logits)
    i2 = jnp.argmax(masked, axis=-1).astype(jnp.int32)
    m2 = jnp.max(masked, axis=-1)
    g1 = jax.nn.sigmoid(m1 - m2)

    e_pair = jnp.stack([i1, i2], axis=-1).reshape(-1)
    t_pair = jnp.repeat(jnp.arange(N, dtype=jnp.int32), K)
    oh = (e_pair[:, None] == jnp.arange(E, dtype=jnp.int32)[None, :]
          ).astype(jnp.int32)
    cum = jnp.cumsum(oh, axis=0)
    counts = cum[-1]
    rank = jnp.sum((cum - oh) * oh, axis=1)

    padded_counts = ((counts + bm - 1) // bm) * bm
    padded_cum = jnp.cumsum(padded_counts)
    padded_offsets = padded_cum - padded_counts
    slot = padded_offsets[e_pair].astype(jnp.int32) + rank

    nb = NP // bm + E
    total = nb * bm

    src_tok = jnp.zeros((total,), jnp.int32).at[slot].set(t_pair)

    block_base = jnp.arange(nb, dtype=jnp.int32) * bm
    e_of_b = jnp.searchsorted(padded_cum, block_base, side="right").astype(jnp.int32)
    e_of_b = jnp.minimum(e_of_b, E - 1)
    active = (block_base < padded_cum[-1]).astype(jnp.int32)

    xs = jnp.take(x_flat.astype(jnp.bfloat16), src_tok, axis=0)

    nf = F // f_blk
    del d_blk
    grid_spec = pltpu.PrefetchScalarGridSpec(
        num_scalar_prefetch=2,
        grid=(nb, nf),
        in_specs=[
            pl.BlockSpec((bm, D), lambda b, f, e_map, act: (b, 0)),
            pl.BlockSpec((1, f_blk, D), lambda b, f, e_map, act: (e_map[b], f, 0)),
            pl.BlockSpec((1, D, f_blk), lambda b, f, e_map, act: (e_map[b], 0, f)),
        ],
        out_specs=pl.BlockSpec((bm, D), lambda b, f, e_map, act: (b, 0)),
    )
    eo = pl.pallas_call(
        _ffn_block_kernel,
        grid_spec=grid_spec,
        out_shape=jax.ShapeDtypeStruct((total, D), jnp.float32),
        compiler_params=pltpu.CompilerParams(
            dimension_semantics=("arbitrary", "arbitrary"),
        ),
    )(e_of_b, active, xs, W1, W2)

    slot_tok = slot.reshape(N, K)
    g1c = g1[:, None]
    out = eo[slot_tok[:, 0]] * g1c + eo[slot_tok[:, 1]] * (1.0 - g1c)
    return out.reshape(B, T, D)


def kernel(x, Wr, W1, W2):
    return _moe(x, Wr, W1, W2, bm=576, f_blk=1024, d_blk=128)
